# Initial kernel scaffold; baseline (speedup 1.0000x reference)
import jax
import jax.numpy as jnp
from jax.experimental import pallas as pl


def kernel(hidden_states, routing_weights, router_indices, gate_up_proj, down_proj):
    raise NotImplementedError("write your pallas kernel here")


# speedup vs baseline: 1.1342x; 1.1342x over previous
import functools

import jax
import jax.numpy as jnp
from jax.experimental import pallas as pl
from jax.experimental.pallas import tpu as pltpu

E = 8
H = 1024
I = 768
T = 2048

TILE_T = 1024


def _moe_body(x_ref, rwt_ref, w1_ref, w2_ref, out_ref):
    e = pl.program_id(1)
    xb = x_ref[...]
    w1 = w1_ref[0].astype(jnp.bfloat16)
    gu = jnp.dot(xb, w1, preferred_element_type=jnp.float32)
    gate = gu[:, :I]
    up = gu[:, I:]
    inter = (up * (gate * jax.nn.sigmoid(gate))).astype(jnp.bfloat16)
    w2 = w2_ref[0].astype(jnp.bfloat16)
    y = jnp.dot(inter, w2, preferred_element_type=jnp.float32)
    y = y * rwt_ref[0, 0][:, None]

    @pl.when(e == 0)
    def _init():
        out_ref[...] = y

    @pl.when(e != 0)
    def _acc():
        out_ref[...] += y


@functools.partial(jax.jit, static_argnames=())
def kernel(hidden_states, routing_weights, router_indices, gate_up_proj, down_proj):
    del router_indices
    x = hidden_states.reshape(T, H).astype(jnp.bfloat16)
    rwt = routing_weights.T.reshape(E, 1, T)

    grid = (T // TILE_T, E)
    out = pl.pallas_call(
        _moe_body,
        grid=grid,
        in_specs=[
            pl.BlockSpec((TILE_T, H), lambda t, e: (t, 0)),
            pl.BlockSpec((1, 1, TILE_T), lambda t, e: (e, 0, t)),
            pl.BlockSpec((1, H, 2 * I), lambda t, e: (e, 0, 0)),
            pl.BlockSpec((1, I, H), lambda t, e: (e, 0, 0)),
        ],
        out_specs=pl.BlockSpec((TILE_T, H), lambda t, e: (t, 0)),
        out_shape=jax.ShapeDtypeStruct((T, H), jnp.float32),
        compiler_params=pltpu.CompilerParams(
            dimension_semantics=("parallel", "arbitrary"),
        ),
    )(x, rwt, gate_up_proj, down_proj)
    return out.reshape(T, 1, H)


# speedup vs baseline: 1.1679x; 1.0297x over previous
import jax
import jax.numpy as jnp
from jax.experimental import pallas as pl
from jax.experimental.pallas import tpu as pltpu

E = 8
H = 1024
I = 768
T = 2048

GROUPS = 2
EPG = E // GROUPS
SUB = 2
TS = T // SUB


def _moe_body(x_ref, rwt_ref, w1_ref, w2_ref, out_ref):
    e = pl.program_id(1)
    w1 = w1_ref[0].astype(jnp.bfloat16)
    w2 = w2_ref[0].astype(jnp.bfloat16)
    for i in range(SUB):
        sl = slice(i * TS, (i + 1) * TS)
        xb = x_ref[sl, :]
        gu = jnp.dot(xb, w1, preferred_element_type=jnp.float32)
        gate = gu[:, :I]
        up = gu[:, I:]
        inter = (up * (gate * jax.nn.sigmoid(gate))).astype(jnp.bfloat16)
        y = jnp.dot(inter, w2, preferred_element_type=jnp.float32)
        y = y * rwt_ref[0, 0, sl][:, None]

        @pl.when(e == 0)
        def _init():
            out_ref[0, sl, :] = y

        @pl.when(e != 0)
        def _acc():
            out_ref[0, sl, :] += y


def kernel(hidden_states, routing_weights, router_indices, gate_up_proj, down_proj):
    del router_indices
    x = hidden_states.reshape(T, H).astype(jnp.bfloat16)
    rwt = routing_weights.T.reshape(E, 1, T)

    grid = (GROUPS, EPG)
    part = pl.pallas_call(
        _moe_body,
        grid=grid,
        in_specs=[
            pl.BlockSpec((T, H), lambda g, e: (0, 0)),
            pl.BlockSpec((1, 1, T), lambda g, e: (g * EPG + e, 0, 0)),
            pl.BlockSpec((1, H, 2 * I), lambda g, e: (g * EPG + e, 0, 0)),
            pl.BlockSpec((1, I, H), lambda g, e: (g * EPG + e, 0, 0)),
        ],
        out_specs=pl.BlockSpec((1, T, H), lambda g, e: (g, 0, 0)),
        out_shape=jax.ShapeDtypeStruct((GROUPS, T, H), jnp.float32),
        compiler_params=pltpu.CompilerParams(
            dimension_semantics=("parallel", "arbitrary"),
        ),
    )(x, rwt, gate_up_proj, down_proj)
    return part.sum(axis=0).reshape(T, 1, H)
